# baseline (speedup 1.0000x reference)
import functools

import jax
import jax.numpy as jnp
from jax import lax
from jax.experimental import pallas as pl
from jax.experimental.pallas import tpu as pltpu
from jax.experimental.pallas import tpu_sc as plsc

_V1, _V2 = 100, 100
_H0, _H1, _H2 = 4, 4, 8
_R1, _R2 = 8, 8
_D = _H0 * _H1 * _H2
_NC, _NS, _L = 2, 16, 16
_NW = _NC * _NS


def _w12_matmul_kernel(a_ref, b_ref, o_ref):
    o_ref[...] = jnp.dot(a_ref[...], b_ref[...],
                         preferred_element_type=jnp.float32)


def _make_sc_gather(n_tokens: int, block: int):
    per_w = n_tokens // _NW
    n_blk = per_w // block
    mesh = plsc.VectorSubcoreMesh(core_axis_name="c", subcore_axis_name="s")

    @functools.partial(
        pl.kernel,
        out_type=jax.ShapeDtypeStruct((n_tokens, _D // 2), jnp.int32),
        mesh=mesh,
        scratch_types=[
            pltpu.VMEM((per_w,), jnp.int32),
            pltpu.VMEM((block,), jnp.int32),
            pltpu.VMEM((block, _D // 2), jnp.int32),
            pltpu.SemaphoreType.DMA,
        ],
        compiler_params=pltpu.CompilerParams(needs_layout_passes=False,
                                             use_tc_tiling_on_sc=False),
    )
    def sc_gather(w12_hbm, ids_hbm, x_hbm, ids_v, idx_v, rows_v, sem):
        wid = lax.axis_index("s") * _NC + lax.axis_index("c")
        base = wid * per_w
        pltpu.sync_copy(ids_hbm.at[pl.ds(base, per_w)], ids_v)

        @pl.loop(0, n_blk)
        def _block(b):
            blk = b * block

            @pl.loop(0, block // _L)
            def _idx(g):
                idv = ids_v[pl.ds(blk + g * _L, _L)]
                idx_v[pl.ds(g * _L, _L)] = lax.div(idv, _V2)

            pltpu.async_copy(w12_hbm.at[idx_v], rows_v, sem).wait()
            pltpu.sync_copy(rows_v, x_hbm.at[pl.ds(base + blk, block)])

    return sc_gather


def _stage2_kernel(ids_ref, x_ref, c2t_ref, o_ref):
    t = x_ref.shape[0]
    idv = ids_ref[0, 0, :]
    i3 = lax.rem(idv, _V2)
    onehot_t = (lax.broadcasted_iota(jnp.int32, (_V2, t), 0) == i3[None, :])
    c3t = jnp.dot(c2t_ref[...], onehot_t.astype(jnp.float32),
                  preferred_element_type=jnp.float32)
    xt = x_ref[...].T.astype(jnp.float32)
    x3 = xt.reshape(_H0 * _H1, _R2, t)
    c33 = c3t.reshape(_R2, _H2, t)
    acc = x3[:, 0, None, :] * c33[None, 0, :, :]
    for r2 in range(1, _R2):
        acc = acc + x3[:, r2, None, :] * c33[None, r2, :, :]
    o_ref[...] = acc.reshape(_D, t).T


def kernel(core0, core1, core2, input_ids):
    b, s = input_ids.shape
    n = b * s

    a = core0.reshape(_V1 * _H0, _R1)
    bm = core1.transpose(1, 0, 2, 3).reshape(_R1, _V2 * _H1 * _R2)
    w_pair = pl.pallas_call(
        _w12_matmul_kernel,
        out_shape=jax.ShapeDtypeStruct((_V1 * _H0, _V2 * _H1 * _R2),
                                       jnp.float32),
    )(a, bm)
    w12 = (w_pair.reshape(_V1, _H0, _V2, _H1 * _R2)
           .transpose(0, 2, 1, 3)
           .reshape(_V1 * _V2, _D))

    c2t = core2.reshape(_V2, _R2 * _H2).T
    ids = input_ids.reshape(n).astype(jnp.int32)

    block = 256
    assert n % (_NW * block) == 0
    w12_packed = lax.bitcast_convert_type(
        w12.astype(jnp.bfloat16).reshape(_V1 * _V2, _D // 2, 2), jnp.int32)
    x_packed = _make_sc_gather(n, block)(w12_packed, ids)
    x = lax.bitcast_convert_type(x_packed, jnp.bfloat16).reshape(n, _D)

    t_blk = 4096
    nb = n // t_blk
    ids3 = ids.reshape(nb, 1, t_blk)
    out = pl.pallas_call(
        _stage2_kernel,
        grid=(nb,),
        in_specs=[
            pl.BlockSpec((1, 1, t_blk), lambda i: (i, 0, 0)),
            pl.BlockSpec((t_blk, _D), lambda i: (i, 0)),
            pl.BlockSpec((_R2 * _H2, _V2), lambda i: (0, 0)),
        ],
        out_specs=pl.BlockSpec((t_blk, _D), lambda i: (i, 0)),
        out_shape=jax.ShapeDtypeStruct((n, _D), jnp.float32),
    )(ids3, x, c2t)
    return out.reshape(b, s, _D)


# speedup vs baseline: 2.4894x; 2.4894x over previous
import functools

import jax
import jax.numpy as jnp
from jax import lax
from jax.experimental import pallas as pl
from jax.experimental.pallas import tpu as pltpu
from jax.experimental.pallas import tpu_sc as plsc

_V1, _V2 = 100, 100
_H0, _H1, _H2 = 4, 4, 8
_R1, _R2 = 8, 8
_D = _H0 * _H1 * _H2
_NC, _NS, _L = 2, 16, 16
_NW = _NC * _NS


def _w12_matmul_kernel(a_ref, b_ref, o_ref):
    o_ref[...] = jnp.dot(a_ref[...], b_ref[...],
                         preferred_element_type=jnp.float32)


def _make_sc_gather(n_tokens: int, block: int):
    per_w = n_tokens // _NW
    n_blk = per_w // block
    mesh = plsc.VectorSubcoreMesh(core_axis_name="c", subcore_axis_name="s")

    @functools.partial(
        pl.kernel,
        out_type=jax.ShapeDtypeStruct((n_tokens, _D), jnp.float32),
        mesh=mesh,
        scratch_types=[
            pltpu.VMEM((per_w,), jnp.int32),
            pltpu.VMEM((block,), jnp.int32),
            pltpu.VMEM((block, _D), jnp.float32),
            pltpu.SemaphoreType.DMA,
        ],
        compiler_params=pltpu.CompilerParams(needs_layout_passes=False),
    )
    def sc_gather(w12_hbm, ids_hbm, x_hbm, ids_v, idx_v, rows_v, sem):
        wid = lax.axis_index("s") * _NC + lax.axis_index("c")
        base = wid * per_w
        pltpu.sync_copy(ids_hbm.at[pl.ds(base, per_w)], ids_v)

        @pl.loop(0, n_blk)
        def _block(b):
            blk = b * block

            @pl.loop(0, block // _L)
            def _idx(g):
                idv = ids_v[pl.ds(blk + g * _L, _L)]
                idx_v[pl.ds(g * _L, _L)] = lax.div(idv, _V2)

            pltpu.async_copy(w12_hbm.at[idx_v], rows_v, sem).wait()
            pltpu.sync_copy(rows_v, x_hbm.at[pl.ds(base + blk, block)])

    return sc_gather


def _stage2_kernel(ids_ref, x_ref, c2t_ref, o_ref):
    t = x_ref.shape[0]
    idv = ids_ref[0, 0, :]
    i3 = lax.rem(idv, _V2)
    onehot_t = (lax.broadcasted_iota(jnp.int32, (_V2, t), 0) == i3[None, :])
    c3t = jnp.dot(c2t_ref[...], onehot_t.astype(jnp.float32),
                  preferred_element_type=jnp.float32)
    xt = x_ref[...].T
    x3 = xt.reshape(_H0 * _H1, _R2, t)
    c33 = c3t.reshape(_R2, _H2, t)
    acc = x3[:, 0, None, :] * c33[None, 0, :, :]
    for r2 in range(1, _R2):
        acc = acc + x3[:, r2, None, :] * c33[None, r2, :, :]
    o_ref[...] = acc.reshape(_D, t).T


def kernel(core0, core1, core2, input_ids):
    b, s = input_ids.shape
    n = b * s

    a = core0.reshape(_V1 * _H0, _R1)
    bm = core1.transpose(1, 0, 2, 3).reshape(_R1, _V2 * _H1 * _R2)
    w_pair = pl.pallas_call(
        _w12_matmul_kernel,
        out_shape=jax.ShapeDtypeStruct((_V1 * _H0, _V2 * _H1 * _R2),
                                       jnp.float32),
    )(a, bm)
    w12 = (w_pair.reshape(_V1, _H0, _V2, _H1 * _R2)
           .transpose(0, 2, 1, 3)
           .reshape(_V1 * _V2, _D))

    c2t = core2.reshape(_V2, _R2 * _H2).T
    ids = input_ids.reshape(n).astype(jnp.int32)

    n_chunks = 5
    nc = n // n_chunks
    block = 256
    t_blk = 4096
    assert nc % (_NW * block) == 0 and nc % t_blk == 0
    sc_gather = _make_sc_gather(nc, block)
    outs = []
    for k in range(n_chunks):
        ids_k = lax.dynamic_slice_in_dim(ids, k * nc, nc)
        x = sc_gather(w12, ids_k)
        nb = nc // t_blk
        ids3 = ids_k.reshape(nb, 1, t_blk)
        outs.append(pl.pallas_call(
            _stage2_kernel,
            grid=(nb,),
            in_specs=[
                pl.BlockSpec((1, 1, t_blk), lambda i: (i, 0, 0)),
                pl.BlockSpec((t_blk, _D), lambda i: (i, 0)),
                pl.BlockSpec((_R2 * _H2, _V2), lambda i: (0, 0)),
            ],
            out_specs=pl.BlockSpec((t_blk, _D), lambda i: (i, 0)),
            out_shape=jax.ShapeDtypeStruct((nc, _D), jnp.float32),
        )(ids3, x, c2t))
    out = jnp.concatenate(outs, axis=0)
    return out.reshape(b, s, _D)


# speedup vs baseline: 2.8965x; 1.1635x over previous
import functools

import jax
import jax.numpy as jnp
from jax import lax
from jax.experimental import pallas as pl
from jax.experimental.pallas import tpu as pltpu
from jax.experimental.pallas import tpu_sc as plsc

_V1, _V2 = 100, 100
_H0, _H1, _H2 = 4, 4, 8
_R1, _R2 = 8, 8
_D = _H0 * _H1 * _H2
_NC, _NS, _L = 2, 16, 16
_NW = _NC * _NS


def _w12_matmul_kernel(a_ref, b_ref, o_ref):
    o_ref[...] = jnp.dot(a_ref[...], b_ref[...],
                         preferred_element_type=jnp.float32)


def _make_sc_gather(n_tokens: int, block: int):
    per_w = n_tokens // _NW
    n_blk = per_w // block
    mesh = plsc.VectorSubcoreMesh(core_axis_name="c", subcore_axis_name="s")

    @functools.partial(
        pl.kernel,
        out_type=jax.ShapeDtypeStruct((n_tokens, _D), jnp.float32),
        mesh=mesh,
        scratch_types=[
            pltpu.VMEM((per_w,), jnp.int32),
            pltpu.VMEM((block,), jnp.int32),
            pltpu.VMEM((block, _D), jnp.float32),
            pltpu.SemaphoreType.DMA,
        ],
        compiler_params=pltpu.CompilerParams(needs_layout_passes=False),
    )
    def sc_gather(w12_hbm, ids_hbm, x_hbm, ids_v, idx_v, rows_v, sem):
        wid = lax.axis_index("s") * _NC + lax.axis_index("c")
        base = wid * per_w
        pltpu.sync_copy(ids_hbm.at[pl.ds(base, per_w)], ids_v)

        @pl.loop(0, n_blk)
        def _block(b):
            blk = b * block

            @pl.loop(0, block // _L)
            def _idx(g):
                idv = ids_v[pl.ds(blk + g * _L, _L)]
                idx_v[pl.ds(g * _L, _L)] = lax.div(idv, _V2)

            pltpu.async_copy(w12_hbm.at[idx_v], rows_v, sem).wait()
            pltpu.sync_copy(rows_v, x_hbm.at[pl.ds(base + blk, block)])

    return sc_gather


def _stage2_kernel(ids_ref, x_ref, c2t_ref, o_ref):
    t = x_ref.shape[0]
    idv = ids_ref[0, 0, :]
    i3 = lax.rem(idv, _V2)
    onehot_t = (lax.broadcasted_iota(jnp.int32, (_V2, t), 0) == i3[None, :])
    c3t = jnp.dot(c2t_ref[...], onehot_t.astype(jnp.float32),
                  preferred_element_type=jnp.float32)
    xt = x_ref[...].T
    x3 = xt.reshape(_H0 * _H1, _R2, t)
    c33 = c3t.reshape(_R2, _H2, t)
    acc = x3[:, 0, None, :] * c33[None, 0, :, :]
    for r2 in range(1, _R2):
        acc = acc + x3[:, r2, None, :] * c33[None, r2, :, :]
    o_ref[...] = acc.reshape(_D, t).T


def kernel(core0, core1, core2, input_ids):
    b, s = input_ids.shape
    n = b * s

    a = core0.reshape(_V1 * _H0, _R1)
    bm = core1.transpose(1, 0, 2, 3).reshape(_R1, _V2 * _H1 * _R2)
    w_pair = pl.pallas_call(
        _w12_matmul_kernel,
        out_shape=jax.ShapeDtypeStruct((_V1 * _H0, _V2 * _H1 * _R2),
                                       jnp.float32),
    )(a, bm)
    w12 = (w_pair.reshape(_V1, _H0, _V2, _H1 * _R2)
           .transpose(0, 2, 1, 3)
           .reshape(_V1 * _V2, _D))

    c2t = core2.reshape(_V2, _R2 * _H2).T
    ids = input_ids.reshape(n).astype(jnp.int32)

    block = 256
    assert n % (_NW * block) == 0
    x = _make_sc_gather(n, block)(w12, ids)

    t_blk = 8192
    nb = n // t_blk
    ids3 = ids.reshape(nb, 1, t_blk)
    out = pl.pallas_call(
        _stage2_kernel,
        grid=(nb,),
        in_specs=[
            pl.BlockSpec((1, 1, t_blk), lambda i: (i, 0, 0)),
            pl.BlockSpec((t_blk, _D), lambda i: (i, 0)),
            pl.BlockSpec((_R2 * _H2, _V2), lambda i: (0, 0)),
        ],
        out_specs=pl.BlockSpec((t_blk, _D), lambda i: (i, 0)),
        out_shape=jax.ShapeDtypeStruct((n, _D), jnp.float32),
    )(ids3, x, c2t)
    return out.reshape(b, s, _D)


# speedup vs baseline: 2.9198x; 1.0081x over previous
import functools

import jax
import jax.numpy as jnp
from jax import lax
from jax.experimental import pallas as pl
from jax.experimental.pallas import tpu as pltpu
from jax.experimental.pallas import tpu_sc as plsc

_V1, _V2 = 100, 100
_H0, _H1, _H2 = 4, 4, 8
_R1, _R2 = 8, 8
_D = _H0 * _H1 * _H2
_NC, _NS, _L = 2, 16, 16
_NW = _NC * _NS


def _w12_matmul_kernel(a_ref, b_ref, o_ref):
    o_ref[...] = jnp.dot(a_ref[...], b_ref[...],
                         preferred_element_type=jnp.float32)


def _make_sc_gather(n_tokens: int, block: int):
    per_w = n_tokens // _NW
    n_blk = per_w // block
    mesh = plsc.VectorSubcoreMesh(core_axis_name="c", subcore_axis_name="s")

    @functools.partial(
        pl.kernel,
        out_type=jax.ShapeDtypeStruct((n_tokens, _D), jnp.float32),
        mesh=mesh,
        scratch_types=[
            pltpu.VMEM((per_w,), jnp.int32),
            pltpu.VMEM((block,), jnp.int32),
            pltpu.VMEM((block, _D), jnp.float32),
            pltpu.SemaphoreType.DMA,
        ],
        compiler_params=pltpu.CompilerParams(needs_layout_passes=False),
    )
    def sc_gather(w12_hbm, ids_hbm, x_hbm, ids_v, idx_v, rows_v, sem):
        wid = lax.axis_index("s") * _NC + lax.axis_index("c")
        base = wid * per_w
        pltpu.sync_copy(ids_hbm.at[pl.ds(base, per_w)], ids_v)

        @pl.loop(0, n_blk)
        def _block(b):
            blk = b * block

            @pl.loop(0, block // _L)
            def _idx(g):
                idv = ids_v[pl.ds(blk + g * _L, _L)]
                idx_v[pl.ds(g * _L, _L)] = lax.div(idv, _V2)

            pltpu.async_copy(w12_hbm.at[idx_v], rows_v, sem).wait()
            pltpu.sync_copy(rows_v, x_hbm.at[pl.ds(base + blk, block)])

    return sc_gather


def _stage2_kernel(ids_ref, x_ref, c2t_ref, o_ref):
    t = x_ref.shape[0]
    idv = ids_ref[0, 0, :]
    i3 = lax.rem(idv, _V2)
    onehot_t = (lax.broadcasted_iota(jnp.int32, (_V2, t), 0) == i3[None, :])
    c3t = jnp.dot(c2t_ref[...], onehot_t.astype(jnp.float32),
                  preferred_element_type=jnp.float32)
    xt = x_ref[...].T
    x3 = xt.reshape(_H0 * _H1, _R2, t)
    c33 = c3t.reshape(_R2, _H2, t)
    acc = x3[:, 0, None, :] * c33[None, 0, :, :]
    for r2 in range(1, _R2):
        acc = acc + x3[:, r2, None, :] * c33[None, r2, :, :]
    o_ref[...] = acc.reshape(_D, t).T


def kernel(core0, core1, core2, input_ids):
    b, s = input_ids.shape
    n = b * s

    a = core0.reshape(_V1 * _H0, _R1)
    bm = core1.transpose(1, 0, 2, 3).reshape(_R1, _V2 * _H1 * _R2)
    w_pair = pl.pallas_call(
        _w12_matmul_kernel,
        out_shape=jax.ShapeDtypeStruct((_V1 * _H0, _V2 * _H1 * _R2),
                                       jnp.float32),
    )(a, bm)
    w12 = (w_pair.reshape(_V1, _H0, _V2, _H1 * _R2)
           .transpose(0, 2, 1, 3)
           .reshape(_V1 * _V2, _D))

    c2t = core2.reshape(_V2, _R2 * _H2).T
    ids = input_ids.reshape(n).astype(jnp.int32)

    block = 256
    assert n % (_NW * block) == 0
    x = _make_sc_gather(n, block)(w12, ids)

    t_blk = 12800
    nb = n // t_blk
    ids3 = ids.reshape(nb, 1, t_blk)
    out = pl.pallas_call(
        _stage2_kernel,
        grid=(nb,),
        in_specs=[
            pl.BlockSpec((1, 1, t_blk), lambda i: (i, 0, 0)),
            pl.BlockSpec((t_blk, _D), lambda i: (i, 0)),
            pl.BlockSpec((_R2 * _H2, _V2), lambda i: (0, 0)),
        ],
        out_specs=pl.BlockSpec((t_blk, _D), lambda i: (i, 0)),
        out_shape=jax.ShapeDtypeStruct((n, _D), jnp.float32),
    )(ids3, x, c2t)
    return out.reshape(b, s, _D)
